# Initial kernel scaffold; baseline (speedup 1.0000x reference)
import jax
import jax.numpy as jnp
from jax.experimental import pallas as pl


def kernel(state, W1, b1, W2, b2, g1, g2, decode_map):
    raise NotImplementedError("write your pallas kernel here")


# speedup vs baseline: 1.0429x; 1.0429x over previous
import jax
import jax.numpy as jnp
from jax.experimental import pallas as pl
from jax.experimental.pallas import tpu as pltpu

_STATE = 128
_HID = 512
_ABS = 2048
_NAG = 32
_APA = 2


def _fused_kernel(state_ref, w1_ref, b1_ref, w2_ref, b2_ref, g1_ref, g2_ref,
                  dm_ref, out_ref):
    x = jnp.maximum(
        jnp.dot(state_ref[...], w1_ref[...],
                preferred_element_type=jnp.float32) + b1_ref[...], 0.0)
    y = jnp.dot(x, w2_ref[...], preferred_element_type=jnp.float32)
    y = y + b2_ref[0] + g1_ref[0] + g2_ref[0]
    m = jnp.max(y, axis=1, keepdims=True)
    lane = jax.lax.broadcasted_iota(jnp.int32, (1, _ABS), 1)
    idx = jnp.min(jnp.where(y == m, lane, _ABS), axis=1, keepdims=True)
    dm = dm_ref[0]
    lane2 = jax.lax.broadcasted_iota(jnp.int32, (_APA, _ABS), 1)
    vals = jnp.sum(jnp.where(lane2 == idx, dm, 0), axis=1, keepdims=True)
    out_ref[...] = jnp.broadcast_to(vals, (_APA, 128)).reshape(1, _APA, 128)


def kernel(state, W1, b1, W2, b2, g1, g2, decode_map):
    state2 = state.reshape(1, _STATE)
    b12 = b1.reshape(1, _HID)
    b2r = b2.reshape(_NAG, 1, _ABS)
    g1r = g1.reshape(_NAG, 1, _ABS)
    g2r = g2.reshape(_NAG, 1, _ABS)
    dm = decode_map.transpose(0, 2, 1)

    out = pl.pallas_call(
        _fused_kernel,
        grid=(_NAG,),
        in_specs=[
            pl.BlockSpec((1, _STATE), lambda i: (0, 0)),
            pl.BlockSpec((_STATE, _HID), lambda i: (0, 0)),
            pl.BlockSpec((1, _HID), lambda i: (0, 0)),
            pl.BlockSpec((_HID, _ABS), lambda i: (0, i)),
            pl.BlockSpec((1, 1, _ABS), lambda i: (i, 0, 0)),
            pl.BlockSpec((1, 1, _ABS), lambda i: (i, 0, 0)),
            pl.BlockSpec((1, 1, _ABS), lambda i: (i, 0, 0)),
            pl.BlockSpec((1, _APA, _ABS), lambda i: (i, 0, 0)),
        ],
        out_specs=pl.BlockSpec((1, _APA, 128), lambda i: (i, 0, 0)),
        out_shape=jax.ShapeDtypeStruct((_NAG, _APA, 128), jnp.int32),
        compiler_params=pltpu.CompilerParams(
            dimension_semantics=("arbitrary",)),
    )(state2, W1, b12, W2, b2r, g1r, g2r, dm)
    return out[:, :, 0].reshape(-1)


# speedup vs baseline: 1.2451x; 1.1939x over previous
import jax
import jax.numpy as jnp
from jax.experimental import pallas as pl
from jax.experimental.pallas import tpu as pltpu

_STATE = 128
_HID = 512
_ABS = 2048
_NAG = 32
_APA = 2


_APS = 2


def _fused_kernel(state_ref, w1_ref, b1_ref, w2_ref, b2_ref, g1_ref, g2_ref,
                  dm_ref, out_ref):
    x = jnp.maximum(
        jnp.dot(state_ref[...], w1_ref[...],
                preferred_element_type=jnp.float32) + b1_ref[...], 0.0)
    y2 = jnp.dot(x, w2_ref[...], preferred_element_type=jnp.float32)
    for a in range(_APS):
        y = y2[:, a * _ABS:(a + 1) * _ABS]
        y = y + b2_ref[a] + g1_ref[a] + g2_ref[a]
        m = jnp.max(y, axis=1, keepdims=True)
        lane = jax.lax.broadcasted_iota(jnp.int32, (1, _ABS), 1)
        idx = jnp.min(jnp.where(y == m, lane, _ABS), axis=1, keepdims=True)
        dm = dm_ref[a]
        lane2 = jax.lax.broadcasted_iota(jnp.int32, (_APA, _ABS), 1)
        vals = jnp.sum(jnp.where(lane2 == idx, dm, 0), axis=1, keepdims=True)
        out_ref[a] = jnp.broadcast_to(vals, (_APA, 128))


def kernel(state, W1, b1, W2, b2, g1, g2, decode_map):
    state2 = state.reshape(1, _STATE)
    b12 = b1.reshape(1, _HID)
    b2r = b2.reshape(_NAG, 1, _ABS)
    g1r = g1.reshape(_NAG, 1, _ABS)
    g2r = g2.reshape(_NAG, 1, _ABS)
    dm = decode_map.transpose(0, 2, 1)

    out = pl.pallas_call(
        _fused_kernel,
        grid=(_NAG // _APS,),
        in_specs=[
            pl.BlockSpec((1, _STATE), lambda i: (0, 0)),
            pl.BlockSpec((_STATE, _HID), lambda i: (0, 0)),
            pl.BlockSpec((1, _HID), lambda i: (0, 0)),
            pl.BlockSpec((_HID, _APS * _ABS), lambda i: (0, i)),
            pl.BlockSpec((_APS, 1, _ABS), lambda i: (i, 0, 0)),
            pl.BlockSpec((_APS, 1, _ABS), lambda i: (i, 0, 0)),
            pl.BlockSpec((_APS, 1, _ABS), lambda i: (i, 0, 0)),
            pl.BlockSpec((_APS, _APA, _ABS), lambda i: (i, 0, 0)),
        ],
        out_specs=pl.BlockSpec((_APS, _APA, 128), lambda i: (i, 0, 0)),
        out_shape=jax.ShapeDtypeStruct((_NAG, _APA, 128), jnp.int32),
        compiler_params=pltpu.CompilerParams(
            dimension_semantics=("arbitrary",)),
    )(state2, W1, b12, W2, b2r, g1r, g2r, dm)
    return out[:, :, 0].reshape(-1)
